# baseline (speedup 1.0000x reference)
import functools

import jax
import jax.numpy as jnp
from jax import lax
from jax.experimental import pallas as pl
from jax.experimental.pallas import tpu as pltpu
from jax.experimental.pallas import tpu_sc as plsc

_N = 10000
_E = 320000
_NC = 2
_NS = 16
_NW = _NC * _NS
_CHUNK = 128
_EP = 327680
_STEPS = _EP // _CHUNK // _NW
_AROWS = _N + 128
_RPT = 640
_RLAST = _N - _RPT
_ZR = 128
_DEGW = 16

_BN = 2000


def _zero_vmem(ref, rows, w):
    zero = jnp.zeros((16,), jnp.float32)

    def body(i, _):
        def inner(j, __):
            ref[i, pl.ds(j * 16, 16)] = zero
            return 0

        return lax.fori_loop(0, w // 16, inner, 0)

    lax.fori_loop(0, rows, body, 0)


def _fill_vmem(ref, rows, w, value):
    val = jnp.full((16,), value, jnp.float32)

    def body(i, _):
        def inner(j, __):
            ref[i, pl.ds(j * 16, 16)] = val
            return 0

        return lax.fori_loop(0, w // 16, inner, 0)

    lax.fori_loop(0, rows, body, 0)


_sc_mesh = plsc.VectorSubcoreMesh(core_axis_name="c", subcore_axis_name="s")


@functools.partial(
    pl.kernel,
    out_type=jax.ShapeDtypeStruct((_NC, _N, _DEGW), jnp.float32),
    mesh=_sc_mesh,
    scratch_types=[
        pltpu.VMEM((_STEPS, _CHUNK), jnp.int32),
        pltpu.VMEM((_CHUNK, _DEGW), jnp.float32),
        pltpu.VMEM((_ZR, _DEGW), jnp.float32),
        pltpu.VMEM_SHARED((_AROWS, _DEGW), jnp.float32),
        pltpu.SemaphoreType.DMA,
    ],
    compiler_params=pltpu.CompilerParams(use_tc_tiling_on_sc=False),
)
def _sc_degree(dst_hbm, out_hbm, dst2, ones, zblk, acc, ssem):
    c = lax.axis_index("c")
    s = lax.axis_index("s")
    row0 = jnp.minimum(s * _RPT, _RLAST)
    tb = (c * _NS + s) * _STEPS
    pltpu.sync_copy(dst_hbm.at[pl.ds(tb, _STEPS)], dst2)
    _zero_vmem(zblk, _ZR, _DEGW)
    _fill_vmem(ones, _CHUNK, _DEGW, 1.0)
    for k in range(_RPT // _ZR):
        pltpu.sync_copy(zblk, acc.at[pl.ds(row0 + k * _ZR, _ZR)])
    plsc.subcore_barrier()

    for i in range(8):
        pltpu.async_copy(ones, acc.at[dst2.at[i]], ssem, add=True)

    def step(i, _):
        pltpu.make_async_copy(ones, acc.at[dst2.at[0]], ssem).wait()
        pltpu.async_copy(ones, acc.at[dst2.at[i]], ssem, add=True)
        return 0

    lax.fori_loop(8, _STEPS, step, 0)
    for _ in range(8):
        pltpu.make_async_copy(ones, acc.at[dst2.at[0]], ssem).wait()
    plsc.subcore_barrier()
    pltpu.sync_copy(acc.at[pl.ds(row0, _RPT)], out_hbm.at[c, pl.ds(row0, _RPT)])


_AGG_CFG = {128: (2, 0, 40), 64: (5, 2, 80), 32: (5, 2, 80), 16: (5, 2, 80)}


def _make_sc_agg(w):
    nbuf, lag, stage = _AGG_CFG[w]
    chunk = _CHUNK
    ahead = nbuf - lag
    passes = _STEPS // stage

    @functools.partial(
        pl.kernel,
        out_type=jax.ShapeDtypeStruct((_NC, _N, w), jnp.float32),
        mesh=_sc_mesh,
        scratch_types=[
            pltpu.VMEM((stage, chunk), jnp.int32),
            pltpu.VMEM((stage, chunk), jnp.int32),
            pltpu.VMEM((nbuf, chunk, w), jnp.float32),
            pltpu.VMEM_SHARED((_AROWS, w), jnp.float32),
            pltpu.SemaphoreType.DMA,
            pltpu.SemaphoreType.DMA,
        ],
        compiler_params=pltpu.CompilerParams(use_tc_tiling_on_sc=False),
    )
    def agg(q_hbm, src_hbm, dst_hbm, out_hbm, src2, dst2, rows, acc, gsem, ssem):
        c = lax.axis_index("c")
        s = lax.axis_index("s")
        row0 = jnp.minimum(s * _RPT, _RLAST)
        tb = (c * _NS + s) * _STEPS

        zero = jnp.zeros((16,), jnp.float32)

        def zi(i, _):
            def zj(j, __):
                rows[0, i, pl.ds(j * 16, 16)] = zero
                return 0

            return lax.fori_loop(0, w // 16, zj, 0)

        lax.fori_loop(0, 80, zi, 0)
        for k in range(_RPT // 80):
            pltpu.sync_copy(
                rows.at[0, pl.ds(0, 80)], acc.at[pl.ds(row0 + k * 80, 80)]
            )
        plsc.subcore_barrier()

        def g_issue(i):
            pltpu.async_copy(q_hbm.at[src2.at[i]], rows.at[lax.rem(i, nbuf)], gsem)

        def g_wait_one():
            pltpu.make_async_copy(q_hbm.at[src2.at[0]], rows.at[0], gsem).wait()

        def s_issue(i):
            pltpu.async_copy(
                rows.at[lax.rem(i, nbuf)], acc.at[dst2.at[i]], ssem, add=True
            )

        def s_wait_one():
            pltpu.make_async_copy(rows.at[0], acc.at[dst2.at[0]], ssem).wait()

        def body(i, _):
            g_wait_one()
            s_issue(i)
            s_wait_one()
            g_issue(i + ahead)
            return 0

        for p in range(passes):
            pltpu.sync_copy(src_hbm.at[pl.ds(tb + p * stage, stage)], src2)
            pltpu.sync_copy(dst_hbm.at[pl.ds(tb + p * stage, stage)], dst2)
            for i in range(ahead):
                g_issue(i)
            for i in range(lag):
                g_wait_one()
                s_issue(i)
                g_issue(i + ahead)
            lax.fori_loop(lag, stage - ahead, body, 0)
            for i in range(stage - ahead, stage):
                g_wait_one()
                s_issue(i)
                s_wait_one()
            for _ in range(lag):
                s_wait_one()
        plsc.subcore_barrier()
        pltpu.sync_copy(acc.at[pl.ds(row0, _RPT)], out_hbm.at[c, pl.ds(row0, _RPT)])

    return agg


_sc_agg = {w: _make_sc_agg(w) for w in (128, 64, 32, 16)}


def _tc_first(degp_ref, x_ref, w_ref, dis_ref, q_ref):
    deg = degp_ref[0, :, 0:1] + degp_ref[1, :, 0:1] + 1.0
    dis = lax.rsqrt(deg)
    dis_ref[...] = dis
    q_ref[...] = jnp.dot(x_ref[...], w_ref[...], preferred_element_type=jnp.float32) * dis


def _tc_mid(part_ref, q_ref, dis_ref, b_ref, w_ref, o_ref):
    s = part_ref[0] + part_ref[1] + q_ref[...]
    h = jnp.maximum(s * dis_ref[...] + b_ref[...], 0.0)
    o_ref[...] = jnp.dot(h, w_ref[...], preferred_element_type=jnp.float32) * dis_ref[...]


def _tc_last(part_ref, q_ref, dis_ref, b_ref, o_ref):
    s = part_ref[0] + part_ref[1] + q_ref[...]
    o_ref[...] = jnp.maximum(s * dis_ref[...] + b_ref[...], 0.0)


def _row_spec(w):
    return pl.BlockSpec((_BN, w), lambda i: (i, 0))


def _part_spec(w):
    return pl.BlockSpec((_NC, _BN, w), lambda i: (0, i, 0))


def _full_spec(a, b):
    return pl.BlockSpec((a, b), lambda i: (0, 0))


def _tc_first_call(degp, x, W):
    return pl.pallas_call(
        _tc_first,
        grid=(_N // _BN,),
        in_specs=[_part_spec(_DEGW), _row_spec(128), _full_spec(128, 128)],
        out_specs=[_row_spec(1), _row_spec(128)],
        out_shape=[
            jax.ShapeDtypeStruct((_N, 1), jnp.float32),
            jax.ShapeDtypeStruct((_N, 128), jnp.float32),
        ],
    )(degp, x, W)


def _tc_mid_call(part, q, dis, b, W):
    w_in, w_out = W.shape
    return pl.pallas_call(
        _tc_mid,
        grid=(_N // _BN,),
        in_specs=[
            _part_spec(w_in),
            _row_spec(w_in),
            _row_spec(1),
            _full_spec(1, w_in),
            _full_spec(w_in, w_out),
        ],
        out_specs=_row_spec(w_out),
        out_shape=jax.ShapeDtypeStruct((_N, w_out), jnp.float32),
    )(part, q, dis, b.reshape(1, -1), W)


def _tc_last_call(part, q, dis, b):
    w = q.shape[1]
    return pl.pallas_call(
        _tc_last,
        grid=(_N // _BN,),
        in_specs=[_part_spec(w), _row_spec(w), _row_spec(1), _full_spec(1, w)],
        out_specs=_row_spec(w),
        out_shape=jax.ShapeDtypeStruct((_N, w), jnp.float32),
    )(part, q, dis, b.reshape(1, -1))


def kernel(x, edge_index, W1, b1, W3, b3, W4, b4, W2, b2):
    npad = _EP - _E
    trash = jnp.stack(
        [jnp.zeros((npad,), jnp.int32), _N + (jnp.arange(npad, dtype=jnp.int32) % 128)]
    )
    ei_p = jnp.concatenate([edge_index, trash], axis=1)
    src2d = ei_p[0].reshape(_EP // _CHUNK, _CHUNK)
    dst2d = ei_p[1].reshape(_EP // _CHUNK, _CHUNK)

    degp = _sc_degree(dst2d)
    dis, q = _tc_first_call(degp, x, W1)

    for b_l, W_next in ((b1, W3), (b3, W4), (b4, W2)):
        part = _sc_agg[q.shape[1]](q, src2d, dst2d)
        q = _tc_mid_call(part, q, dis, b_l, W_next)

    part = _sc_agg[16](q, src2d, dst2d)
    return _tc_last_call(part, q, dis, b2)


# speedup vs baseline: 2.5308x; 2.5308x over previous
import functools

import jax
import jax.numpy as jnp
from jax import lax
from jax.experimental import pallas as pl
from jax.experimental.pallas import tpu as pltpu
from jax.experimental.pallas import tpu_sc as plsc

_N = 10000
_E = 320000
_NC = 2
_NS = 16
_NW = _NC * _NS
_CHUNK = 128
_EP = 327680
_STEPS = _EP // _CHUNK // _NW
_AROWS = _N + 128
_RPT = 640
_RLAST = _N - _RPT
_ZR = 128
_DEGW = 16

_BN = 2000


def _zero_vmem(ref, rows, w):
    zero = jnp.zeros((16,), jnp.float32)

    def body(i, _):
        def inner(j, __):
            ref[i, pl.ds(j * 16, 16)] = zero
            return 0

        return lax.fori_loop(0, w // 16, inner, 0)

    lax.fori_loop(0, rows, body, 0)


def _fill_vmem(ref, rows, w, value):
    val = jnp.full((16,), value, jnp.float32)

    def body(i, _):
        def inner(j, __):
            ref[i, pl.ds(j * 16, 16)] = val
            return 0

        return lax.fori_loop(0, w // 16, inner, 0)

    lax.fori_loop(0, rows, body, 0)


_sc_mesh = plsc.VectorSubcoreMesh(core_axis_name="c", subcore_axis_name="s")


@functools.partial(
    pl.kernel,
    out_type=jax.ShapeDtypeStruct((_NC, _N, _DEGW), jnp.float32),
    mesh=_sc_mesh,
    scratch_types=[
        pltpu.VMEM((_STEPS, _CHUNK), jnp.int32),
        pltpu.VMEM((_CHUNK, _DEGW), jnp.float32),
        pltpu.VMEM((_ZR, _DEGW), jnp.float32),
        pltpu.VMEM_SHARED((_AROWS, _DEGW), jnp.float32),
        pltpu.SemaphoreType.DMA,
    ],
    compiler_params=pltpu.CompilerParams(use_tc_tiling_on_sc=False),
)
def _sc_degree(dst_hbm, out_hbm, dst2, ones, zblk, acc, ssem):
    c = lax.axis_index("c")
    s = lax.axis_index("s")
    row0 = jnp.minimum(s * _RPT, _RLAST)
    tb = (c * _NS + s) * _STEPS
    pltpu.sync_copy(dst_hbm.at[pl.ds(tb, _STEPS)], dst2)
    _zero_vmem(zblk, _ZR, _DEGW)
    _fill_vmem(ones, _CHUNK, _DEGW, 1.0)
    for k in range(_RPT // _ZR):
        pltpu.sync_copy(zblk, acc.at[pl.ds(row0 + k * _ZR, _ZR)])
    plsc.subcore_barrier()

    for i in range(8):
        pltpu.async_copy(ones, acc.at[dst2.at[i]], ssem, add=True)

    def step(i, _):
        pltpu.make_async_copy(ones, acc.at[dst2.at[0]], ssem).wait()
        pltpu.async_copy(ones, acc.at[dst2.at[i]], ssem, add=True)
        return 0

    lax.fori_loop(8, _STEPS, step, 0)
    for _ in range(8):
        pltpu.make_async_copy(ones, acc.at[dst2.at[0]], ssem).wait()
    plsc.subcore_barrier()
    pltpu.sync_copy(acc.at[pl.ds(row0, _RPT)], out_hbm.at[c, pl.ds(row0, _RPT)])


_AGG_CFG = {128: (2, 0, 40), 64: (5, 2, 80), 32: (5, 2, 80), 16: (5, 2, 80)}


def _make_sc_agg(w):
    nbuf, lag, stage = _AGG_CFG[w]
    chunk = _CHUNK
    ahead = nbuf - lag
    passes = _STEPS // stage

    @functools.partial(
        pl.kernel,
        out_type=jax.ShapeDtypeStruct((_NC, _N, w), jnp.float32),
        mesh=_sc_mesh,
        scratch_types=[
            pltpu.VMEM((stage, chunk), jnp.int32),
            pltpu.VMEM((stage, chunk), jnp.int32),
            pltpu.VMEM((nbuf, chunk, w), jnp.float32),
            pltpu.VMEM_SHARED((_AROWS, w), jnp.float32),
            pltpu.SemaphoreType.DMA,
            pltpu.SemaphoreType.DMA,
        ],
        compiler_params=pltpu.CompilerParams(use_tc_tiling_on_sc=False),
    )
    def agg(q_hbm, src_hbm, dst_hbm, out_hbm, src2, dst2, rows, acc, gsem, ssem):
        c = lax.axis_index("c")
        s = lax.axis_index("s")
        row0 = jnp.minimum(s * _RPT, _RLAST)
        tb = (c * _NS + s) * _STEPS

        zero = jnp.zeros((16,), jnp.float32)

        def zi(i, _):
            def zj(j, __):
                rows[0, i, pl.ds(j * 16, 16)] = zero
                return 0

            return lax.fori_loop(0, w // 16, zj, 0)

        lax.fori_loop(0, 80, zi, 0)
        for k in range(_RPT // 80):
            pltpu.sync_copy(
                rows.at[0, pl.ds(0, 80)], acc.at[pl.ds(row0 + k * 80, 80)]
            )
        plsc.subcore_barrier()

        def g_issue(i):
            pltpu.async_copy(q_hbm.at[src2.at[i]], rows.at[lax.rem(i, nbuf)], gsem)

        def g_wait_one():
            pltpu.make_async_copy(q_hbm.at[src2.at[0]], rows.at[0], gsem).wait()

        def s_issue(i):
            pltpu.async_copy(
                rows.at[lax.rem(i, nbuf)], acc.at[dst2.at[i]], ssem, add=True
            )

        def s_wait_one():
            pltpu.make_async_copy(rows.at[0], acc.at[dst2.at[0]], ssem).wait()

        def body(i, _):
            g_wait_one()
            s_issue(i)
            s_wait_one()
            g_issue(i + ahead)
            return 0

        for p in range(passes):
            pltpu.sync_copy(src_hbm.at[pl.ds(tb + p * stage, stage)], src2)
            pltpu.sync_copy(dst_hbm.at[pl.ds(tb + p * stage, stage)], dst2)
            for i in range(ahead):
                g_issue(i)
            for i in range(lag):
                g_wait_one()
                s_issue(i)
                g_issue(i + ahead)
            lax.fori_loop(lag, stage - ahead, body, 0)
            for i in range(stage - ahead, stage):
                g_wait_one()
                s_issue(i)
                s_wait_one()
            for _ in range(lag):
                s_wait_one()
        plsc.subcore_barrier()
        pltpu.sync_copy(acc.at[pl.ds(row0, _RPT)], out_hbm.at[c, pl.ds(row0, _RPT)])

    return agg


_sc_agg = {w: _make_sc_agg(w) for w in (128, 64, 32, 16)}


def _tc_first(degp_ref, x_ref, w_ref, dis_ref, q_ref):
    deg = degp_ref[0, :, 0:1] + degp_ref[1, :, 0:1] + 1.0
    dis = lax.rsqrt(deg)
    dis_ref[...] = dis
    q_ref[...] = jnp.dot(x_ref[...], w_ref[...], preferred_element_type=jnp.float32) * dis


def _tc_mid(part_ref, q_ref, dis_ref, b_ref, w_ref, o_ref):
    s = part_ref[0] + part_ref[1] + q_ref[...]
    h = jnp.maximum(s * dis_ref[...] + b_ref[...], 0.0)
    o_ref[...] = jnp.dot(h, w_ref[...], preferred_element_type=jnp.float32) * dis_ref[...]


def _tc_last(part_ref, q_ref, dis_ref, b_ref, o_ref):
    s = part_ref[0] + part_ref[1] + q_ref[...]
    o_ref[...] = jnp.maximum(s * dis_ref[...] + b_ref[...], 0.0)


def _row_spec(w):
    return pl.BlockSpec((_BN, w), lambda i: (i, 0))


def _part_spec(w):
    return pl.BlockSpec((_NC, _BN, w), lambda i: (0, i, 0))


def _full_spec(a, b):
    return pl.BlockSpec((a, b), lambda i: (0, 0))


def _tc_first_call(degp, x, W):
    return pl.pallas_call(
        _tc_first,
        grid=(_N // _BN,),
        in_specs=[_part_spec(_DEGW), _row_spec(128), _full_spec(128, 128)],
        out_specs=[_row_spec(1), _row_spec(128)],
        out_shape=[
            jax.ShapeDtypeStruct((_N, 1), jnp.float32),
            jax.ShapeDtypeStruct((_N, 128), jnp.float32),
        ],
    )(degp, x, W)


def _tc_mid_call(part, q, dis, b, W):
    w_in, w_out = W.shape
    return pl.pallas_call(
        _tc_mid,
        grid=(_N // _BN,),
        in_specs=[
            _part_spec(w_in),
            _row_spec(w_in),
            _row_spec(1),
            _full_spec(1, w_in),
            _full_spec(w_in, w_out),
        ],
        out_specs=_row_spec(w_out),
        out_shape=jax.ShapeDtypeStruct((_N, w_out), jnp.float32),
    )(part, q, dis, b.reshape(1, -1), W)


def _tc_last_call(part, q, dis, b):
    w = q.shape[1]
    return pl.pallas_call(
        _tc_last,
        grid=(_N // _BN,),
        in_specs=[_part_spec(w), _row_spec(w), _row_spec(1), _full_spec(1, w)],
        out_specs=_row_spec(w),
        out_shape=jax.ShapeDtypeStruct((_N, w), jnp.float32),
    )(part, q, dis, b.reshape(1, -1))


def kernel(x, edge_index, W1, b1, W3, b3, W4, b4, W2, b2):
    ppt = (_EP - _E) // _NW
    cyc = jnp.arange(ppt, dtype=jnp.int32) % 128
    trash = jnp.broadcast_to(
        jnp.stack([cyc, _N + cyc])[:, None, :], (2, _NW, ppt)
    )
    ei_p = jnp.concatenate(
        [edge_index.reshape(2, _NW, _E // _NW), trash], axis=2
    )
    src2d = ei_p[0].reshape(_EP // _CHUNK, _CHUNK)
    dst2d = ei_p[1].reshape(_EP // _CHUNK, _CHUNK)

    degp = _sc_degree(dst2d)
    dis, q = _tc_first_call(degp, x, W1)

    for b_l, W_next in ((b1, W3), (b3, W4), (b4, W2)):
        part = _sc_agg[q.shape[1]](q, src2d, dst2d)
        q = _tc_mid_call(part, q, dis, b_l, W_next)

    part = _sc_agg[16](q, src2d, dst2d)
    return _tc_last_call(part, q, dis, b2)


# speedup vs baseline: 2.5627x; 1.0126x over previous
import functools

import jax
import jax.numpy as jnp
from jax import lax
from jax.experimental import pallas as pl
from jax.experimental.pallas import tpu as pltpu
from jax.experimental.pallas import tpu_sc as plsc

_N = 10000
_E = 320000
_NC = 2
_NS = 16
_NW = _NC * _NS
_CHUNK = 128
_EP = 327680
_STEPS = _EP // _CHUNK // _NW
_AROWS = _N + 128
_RPT = 640
_RLAST = _N - _RPT
_ZR = 128
_DEGW = 16

_BN = 2000


def _zero_vmem(ref, rows, w):
    zero = jnp.zeros((16,), jnp.float32)

    def body(i, _):
        def inner(j, __):
            ref[i, pl.ds(j * 16, 16)] = zero
            return 0

        return lax.fori_loop(0, w // 16, inner, 0)

    lax.fori_loop(0, rows, body, 0)


def _fill_vmem(ref, rows, w, value):
    val = jnp.full((16,), value, jnp.float32)

    def body(i, _):
        def inner(j, __):
            ref[i, pl.ds(j * 16, 16)] = val
            return 0

        return lax.fori_loop(0, w // 16, inner, 0)

    lax.fori_loop(0, rows, body, 0)


_sc_mesh = plsc.VectorSubcoreMesh(core_axis_name="c", subcore_axis_name="s")


@functools.partial(
    pl.kernel,
    out_type=jax.ShapeDtypeStruct((_NC, _N, _DEGW), jnp.float32),
    mesh=_sc_mesh,
    scratch_types=[
        pltpu.VMEM((_STEPS, _CHUNK), jnp.int32),
        pltpu.VMEM((_CHUNK, _DEGW), jnp.float32),
        pltpu.VMEM((_ZR, _DEGW), jnp.float32),
        pltpu.VMEM_SHARED((_AROWS, _DEGW), jnp.float32),
        pltpu.SemaphoreType.DMA,
    ],
    compiler_params=pltpu.CompilerParams(use_tc_tiling_on_sc=False),
)
def _sc_degree(dst_hbm, out_hbm, dst2, ones, zblk, acc, ssem):
    c = lax.axis_index("c")
    s = lax.axis_index("s")
    row0 = jnp.minimum(s * _RPT, _RLAST)
    tb = (c * _NS + s) * _STEPS
    pltpu.sync_copy(dst_hbm.at[pl.ds(tb, _STEPS)], dst2)
    _zero_vmem(zblk, _ZR, _DEGW)
    _fill_vmem(ones, _CHUNK, _DEGW, 1.0)
    for k in range(_RPT // _ZR):
        pltpu.sync_copy(zblk, acc.at[pl.ds(row0 + k * _ZR, _ZR)])
    plsc.subcore_barrier()

    for i in range(8):
        pltpu.async_copy(ones, acc.at[dst2.at[i]], ssem, add=True)

    def step(i, _):
        pltpu.make_async_copy(ones, acc.at[dst2.at[0]], ssem).wait()
        pltpu.async_copy(ones, acc.at[dst2.at[i]], ssem, add=True)
        return 0

    lax.fori_loop(8, _STEPS, step, 0)
    for _ in range(8):
        pltpu.make_async_copy(ones, acc.at[dst2.at[0]], ssem).wait()
    plsc.subcore_barrier()
    pltpu.sync_copy(acc.at[pl.ds(row0, _RPT)], out_hbm.at[c, pl.ds(row0, _RPT)])


_AGG_CFG = {128: (2, 0, 40), 64: (5, 2, 80), 32: (5, 2, 80), 16: (8, 4, 80)}


def _make_sc_agg(w):
    nbuf, lag, stage = _AGG_CFG[w]
    chunk = _CHUNK
    ahead = nbuf - lag
    passes = _STEPS // stage

    @functools.partial(
        pl.kernel,
        out_type=jax.ShapeDtypeStruct((_NC, _N, w), jnp.float32),
        mesh=_sc_mesh,
        scratch_types=[
            pltpu.VMEM((stage, chunk), jnp.int32),
            pltpu.VMEM((stage, chunk), jnp.int32),
            pltpu.VMEM((nbuf, chunk, w), jnp.float32),
            pltpu.VMEM_SHARED((_AROWS, w), jnp.float32),
            pltpu.SemaphoreType.DMA,
            pltpu.SemaphoreType.DMA,
        ],
        compiler_params=pltpu.CompilerParams(use_tc_tiling_on_sc=False),
    )
    def agg(q_hbm, src_hbm, dst_hbm, out_hbm, src2, dst2, rows, acc, gsem, ssem):
        c = lax.axis_index("c")
        s = lax.axis_index("s")
        row0 = jnp.minimum(s * _RPT, _RLAST)
        tb = (c * _NS + s) * _STEPS

        zero = jnp.zeros((16,), jnp.float32)

        def zi(i, _):
            def zj(j, __):
                rows[0, i, pl.ds(j * 16, 16)] = zero
                return 0

            return lax.fori_loop(0, w // 16, zj, 0)

        lax.fori_loop(0, 80, zi, 0)
        for k in range(_RPT // 80):
            pltpu.sync_copy(
                rows.at[0, pl.ds(0, 80)], acc.at[pl.ds(row0 + k * 80, 80)]
            )
        plsc.subcore_barrier()

        def g_issue(i):
            pltpu.async_copy(q_hbm.at[src2.at[i]], rows.at[lax.rem(i, nbuf)], gsem)

        def g_wait_one():
            pltpu.make_async_copy(q_hbm.at[src2.at[0]], rows.at[0], gsem).wait()

        def s_issue(i):
            pltpu.async_copy(
                rows.at[lax.rem(i, nbuf)], acc.at[dst2.at[i]], ssem, add=True
            )

        def s_wait_one():
            pltpu.make_async_copy(rows.at[0], acc.at[dst2.at[0]], ssem).wait()

        def body(i, _):
            g_wait_one()
            s_issue(i)
            s_wait_one()
            g_issue(i + ahead)
            return 0

        for p in range(passes):
            pltpu.sync_copy(src_hbm.at[pl.ds(tb + p * stage, stage)], src2)
            pltpu.sync_copy(dst_hbm.at[pl.ds(tb + p * stage, stage)], dst2)
            for i in range(ahead):
                g_issue(i)
            for i in range(lag):
                g_wait_one()
                s_issue(i)
                g_issue(i + ahead)
            lax.fori_loop(lag, stage - ahead, body, 0)
            for i in range(stage - ahead, stage):
                g_wait_one()
                s_issue(i)
                s_wait_one()
            for _ in range(lag):
                s_wait_one()
        plsc.subcore_barrier()
        pltpu.sync_copy(acc.at[pl.ds(row0, _RPT)], out_hbm.at[c, pl.ds(row0, _RPT)])

    return agg


_sc_agg = {w: _make_sc_agg(w) for w in (128, 64, 32, 16)}


def _tc_first(degp_ref, x_ref, w_ref, dis_ref, q_ref):
    deg = degp_ref[0, :, 0:1] + degp_ref[1, :, 0:1] + 1.0
    dis = lax.rsqrt(deg)
    dis_ref[...] = dis
    q_ref[...] = jnp.dot(x_ref[...], w_ref[...], preferred_element_type=jnp.float32) * dis


def _tc_mid(part_ref, q_ref, dis_ref, b_ref, w_ref, o_ref):
    s = part_ref[0] + part_ref[1] + q_ref[...]
    h = jnp.maximum(s * dis_ref[...] + b_ref[...], 0.0)
    o_ref[...] = jnp.dot(h, w_ref[...], preferred_element_type=jnp.float32) * dis_ref[...]


def _tc_last(part_ref, q_ref, dis_ref, b_ref, o_ref):
    s = part_ref[0] + part_ref[1] + q_ref[...]
    o_ref[...] = jnp.maximum(s * dis_ref[...] + b_ref[...], 0.0)


def _row_spec(w):
    return pl.BlockSpec((_BN, w), lambda i: (i, 0))


def _part_spec(w):
    return pl.BlockSpec((_NC, _BN, w), lambda i: (0, i, 0))


def _full_spec(a, b):
    return pl.BlockSpec((a, b), lambda i: (0, 0))


def _tc_first_call(degp, x, W):
    return pl.pallas_call(
        _tc_first,
        grid=(_N // _BN,),
        in_specs=[_part_spec(_DEGW), _row_spec(128), _full_spec(128, 128)],
        out_specs=[_row_spec(1), _row_spec(128)],
        out_shape=[
            jax.ShapeDtypeStruct((_N, 1), jnp.float32),
            jax.ShapeDtypeStruct((_N, 128), jnp.float32),
        ],
    )(degp, x, W)


def _tc_mid_call(part, q, dis, b, W):
    w_in, w_out = W.shape
    return pl.pallas_call(
        _tc_mid,
        grid=(_N // _BN,),
        in_specs=[
            _part_spec(w_in),
            _row_spec(w_in),
            _row_spec(1),
            _full_spec(1, w_in),
            _full_spec(w_in, w_out),
        ],
        out_specs=_row_spec(w_out),
        out_shape=jax.ShapeDtypeStruct((_N, w_out), jnp.float32),
    )(part, q, dis, b.reshape(1, -1), W)


def _tc_last_call(part, q, dis, b):
    w = q.shape[1]
    return pl.pallas_call(
        _tc_last,
        grid=(_N // _BN,),
        in_specs=[_part_spec(w), _row_spec(w), _row_spec(1), _full_spec(1, w)],
        out_specs=_row_spec(w),
        out_shape=jax.ShapeDtypeStruct((_N, w), jnp.float32),
    )(part, q, dis, b.reshape(1, -1))


def kernel(x, edge_index, W1, b1, W3, b3, W4, b4, W2, b2):
    ppt = (_EP - _E) // _NW
    cyc = jnp.arange(ppt, dtype=jnp.int32) % 128
    trash = jnp.broadcast_to(
        jnp.stack([cyc, _N + cyc])[:, None, :], (2, _NW, ppt)
    )
    ei_p = jnp.concatenate(
        [edge_index.reshape(2, _NW, _E // _NW), trash], axis=2
    )
    src2d = ei_p[0].reshape(_EP // _CHUNK, _CHUNK)
    dst2d = ei_p[1].reshape(_EP // _CHUNK, _CHUNK)

    degp = _sc_degree(dst2d)
    dis, q = _tc_first_call(degp, x, W1)

    for b_l, W_next in ((b1, W3), (b3, W4), (b4, W2)):
        part = _sc_agg[q.shape[1]](q, src2d, dst2d)
        q = _tc_mid_call(part, q, dis, b_l, W_next)

    part = _sc_agg[16](q, src2d, dst2d)
    return _tc_last_call(part, q, dis, b2)


# speedup vs baseline: 2.5867x; 1.0094x over previous
import functools

import jax
import jax.numpy as jnp
from jax import lax
from jax.experimental import pallas as pl
from jax.experimental.pallas import tpu as pltpu
from jax.experimental.pallas import tpu_sc as plsc

_N = 10000
_E = 320000
_NC = 2
_NS = 16
_NW = _NC * _NS
_CHUNK = 128
_EP = 327680
_STEPS = _EP // _CHUNK // _NW
_AROWS = _N + 128
_RPT = 640
_RLAST = _N - _RPT
_ZR = 128
_DEGW = 16

_BN = 2000


def _zero_vmem(ref, rows, w):
    zero = jnp.zeros((16,), jnp.float32)

    def body(i, _):
        def inner(j, __):
            ref[i, pl.ds(j * 16, 16)] = zero
            return 0

        return lax.fori_loop(0, w // 16, inner, 0)

    lax.fori_loop(0, rows, body, 0)


def _fill_vmem(ref, rows, w, value):
    val = jnp.full((16,), value, jnp.float32)

    def body(i, _):
        def inner(j, __):
            ref[i, pl.ds(j * 16, 16)] = val
            return 0

        return lax.fori_loop(0, w // 16, inner, 0)

    lax.fori_loop(0, rows, body, 0)


_sc_mesh = plsc.VectorSubcoreMesh(core_axis_name="c", subcore_axis_name="s")


@functools.partial(
    pl.kernel,
    out_type=jax.ShapeDtypeStruct((_NC, _N, _DEGW), jnp.float32),
    mesh=_sc_mesh,
    scratch_types=[
        pltpu.VMEM((_STEPS, _CHUNK), jnp.int32),
        pltpu.VMEM((_CHUNK, _DEGW), jnp.float32),
        pltpu.VMEM((_ZR, _DEGW), jnp.float32),
        pltpu.VMEM_SHARED((_AROWS, _DEGW), jnp.float32),
        pltpu.SemaphoreType.DMA,
    ],
    compiler_params=pltpu.CompilerParams(use_tc_tiling_on_sc=False),
)
def _sc_degree(dst_hbm, out_hbm, dst2, ones, zblk, acc, ssem):
    c = lax.axis_index("c")
    s = lax.axis_index("s")
    row0 = jnp.minimum(s * _RPT, _RLAST)
    tb = (c * _NS + s) * _STEPS
    pltpu.sync_copy(dst_hbm.at[pl.ds(tb, _STEPS)], dst2)
    _zero_vmem(zblk, _ZR, _DEGW)
    _fill_vmem(ones, _CHUNK, _DEGW, 1.0)
    for k in range(_RPT // _ZR):
        pltpu.sync_copy(zblk, acc.at[pl.ds(row0 + k * _ZR, _ZR)])
    plsc.subcore_barrier()

    for i in range(8):
        pltpu.async_copy(ones, acc.at[dst2.at[i]], ssem, add=True)

    def step(i, _):
        pltpu.make_async_copy(ones, acc.at[dst2.at[0]], ssem).wait()
        pltpu.async_copy(ones, acc.at[dst2.at[i]], ssem, add=True)
        return 0

    lax.fori_loop(8, _STEPS, step, 0)
    for _ in range(8):
        pltpu.make_async_copy(ones, acc.at[dst2.at[0]], ssem).wait()
    plsc.subcore_barrier()
    pltpu.sync_copy(acc.at[pl.ds(row0, _RPT)], out_hbm.at[c, pl.ds(row0, _RPT)])


_AGG_CFG = {128: (2, 0, 40), 64: (8, 4, 80), 32: (8, 4, 80), 16: (8, 4, 80)}


def _make_sc_agg(w):
    nbuf, lag, stage = _AGG_CFG[w]
    chunk = _CHUNK
    ahead = nbuf - lag
    passes = _STEPS // stage

    @functools.partial(
        pl.kernel,
        out_type=jax.ShapeDtypeStruct((_NC, _N, w), jnp.float32),
        mesh=_sc_mesh,
        scratch_types=[
            pltpu.VMEM((stage, chunk), jnp.int32),
            pltpu.VMEM((stage, chunk), jnp.int32),
            pltpu.VMEM((nbuf, chunk, w), jnp.float32),
            pltpu.VMEM_SHARED((_AROWS, w), jnp.float32),
            pltpu.SemaphoreType.DMA,
            pltpu.SemaphoreType.DMA,
        ],
        compiler_params=pltpu.CompilerParams(use_tc_tiling_on_sc=False),
    )
    def agg(q_hbm, src_hbm, dst_hbm, out_hbm, src2, dst2, rows, acc, gsem, ssem):
        c = lax.axis_index("c")
        s = lax.axis_index("s")
        row0 = jnp.minimum(s * _RPT, _RLAST)
        tb = (c * _NS + s) * _STEPS

        zero = jnp.zeros((16,), jnp.float32)

        def zi(i, _):
            def zj(j, __):
                rows[0, i, pl.ds(j * 16, 16)] = zero
                return 0

            return lax.fori_loop(0, w // 16, zj, 0)

        lax.fori_loop(0, 80, zi, 0)
        for k in range(_RPT // 80):
            pltpu.sync_copy(
                rows.at[0, pl.ds(0, 80)], acc.at[pl.ds(row0 + k * 80, 80)]
            )
        plsc.subcore_barrier()

        def g_issue(i):
            pltpu.async_copy(q_hbm.at[src2.at[i]], rows.at[lax.rem(i, nbuf)], gsem)

        def g_wait_one():
            pltpu.make_async_copy(q_hbm.at[src2.at[0]], rows.at[0], gsem).wait()

        def s_issue(i):
            pltpu.async_copy(
                rows.at[lax.rem(i, nbuf)], acc.at[dst2.at[i]], ssem, add=True
            )

        def s_wait_one():
            pltpu.make_async_copy(rows.at[0], acc.at[dst2.at[0]], ssem).wait()

        def body(i, _):
            g_wait_one()
            s_issue(i)
            s_wait_one()
            g_issue(i + ahead)
            return 0

        for p in range(passes):
            pltpu.sync_copy(src_hbm.at[pl.ds(tb + p * stage, stage)], src2)
            pltpu.sync_copy(dst_hbm.at[pl.ds(tb + p * stage, stage)], dst2)
            for i in range(ahead):
                g_issue(i)
            for i in range(lag):
                g_wait_one()
                s_issue(i)
                g_issue(i + ahead)
            lax.fori_loop(lag, stage - ahead, body, 0)
            for i in range(stage - ahead, stage):
                g_wait_one()
                s_issue(i)
                s_wait_one()
            for _ in range(lag):
                s_wait_one()
        plsc.subcore_barrier()
        pltpu.sync_copy(acc.at[pl.ds(row0, _RPT)], out_hbm.at[c, pl.ds(row0, _RPT)])

    return agg


_sc_agg = {w: _make_sc_agg(w) for w in (128, 64, 32, 16)}


def _tc_first(degp_ref, x_ref, w_ref, dis_ref, q_ref):
    deg = degp_ref[0, :, 0:1] + degp_ref[1, :, 0:1] + 1.0
    dis = lax.rsqrt(deg)
    dis_ref[...] = dis
    q_ref[...] = jnp.dot(x_ref[...], w_ref[...], preferred_element_type=jnp.float32) * dis


def _tc_mid(part_ref, q_ref, dis_ref, b_ref, w_ref, o_ref):
    s = part_ref[0] + part_ref[1] + q_ref[...]
    h = jnp.maximum(s * dis_ref[...] + b_ref[...], 0.0)
    o_ref[...] = jnp.dot(h, w_ref[...], preferred_element_type=jnp.float32) * dis_ref[...]


def _tc_last(part_ref, q_ref, dis_ref, b_ref, o_ref):
    s = part_ref[0] + part_ref[1] + q_ref[...]
    o_ref[...] = jnp.maximum(s * dis_ref[...] + b_ref[...], 0.0)


def _row_spec(w):
    return pl.BlockSpec((_BN, w), lambda i: (i, 0))


def _part_spec(w):
    return pl.BlockSpec((_NC, _BN, w), lambda i: (0, i, 0))


def _full_spec(a, b):
    return pl.BlockSpec((a, b), lambda i: (0, 0))


def _tc_first_call(degp, x, W):
    return pl.pallas_call(
        _tc_first,
        grid=(_N // _BN,),
        in_specs=[_part_spec(_DEGW), _row_spec(128), _full_spec(128, 128)],
        out_specs=[_row_spec(1), _row_spec(128)],
        out_shape=[
            jax.ShapeDtypeStruct((_N, 1), jnp.float32),
            jax.ShapeDtypeStruct((_N, 128), jnp.float32),
        ],
    )(degp, x, W)


def _tc_mid_call(part, q, dis, b, W):
    w_in, w_out = W.shape
    return pl.pallas_call(
        _tc_mid,
        grid=(_N // _BN,),
        in_specs=[
            _part_spec(w_in),
            _row_spec(w_in),
            _row_spec(1),
            _full_spec(1, w_in),
            _full_spec(w_in, w_out),
        ],
        out_specs=_row_spec(w_out),
        out_shape=jax.ShapeDtypeStruct((_N, w_out), jnp.float32),
    )(part, q, dis, b.reshape(1, -1), W)


def _tc_last_call(part, q, dis, b):
    w = q.shape[1]
    return pl.pallas_call(
        _tc_last,
        grid=(_N // _BN,),
        in_specs=[_part_spec(w), _row_spec(w), _row_spec(1), _full_spec(1, w)],
        out_specs=_row_spec(w),
        out_shape=jax.ShapeDtypeStruct((_N, w), jnp.float32),
    )(part, q, dis, b.reshape(1, -1))


def kernel(x, edge_index, W1, b1, W3, b3, W4, b4, W2, b2):
    ppt = (_EP - _E) // _NW
    cyc = jnp.arange(ppt, dtype=jnp.int32) % 128
    trash = jnp.broadcast_to(
        jnp.stack([cyc, _N + cyc])[:, None, :], (2, _NW, ppt)
    )
    ei_p = jnp.concatenate(
        [edge_index.reshape(2, _NW, _E // _NW), trash], axis=2
    )
    src2d = ei_p[0].reshape(_EP // _CHUNK, _CHUNK)
    dst2d = ei_p[1].reshape(_EP // _CHUNK, _CHUNK)

    degp = _sc_degree(dst2d)
    dis, q = _tc_first_call(degp, x, W1)

    for b_l, W_next in ((b1, W3), (b3, W4), (b4, W2)):
        part = _sc_agg[q.shape[1]](q, src2d, dst2d)
        q = _tc_mid_call(part, q, dis, b_l, W_next)

    part = _sc_agg[16](q, src2d, dst2d)
    return _tc_last_call(part, q, dis, b2)
